# Initial kernel scaffold; baseline (speedup 1.0000x reference)
import jax
import jax.numpy as jnp
from jax.experimental import pallas as pl


def kernel(ploc, pconf, gloc, glabel, dboxes):
    raise NotImplementedError("write your pallas kernel here")


# speedup vs baseline: 6.3373x; 6.3373x over previous
import functools

import jax
import jax.numpy as jnp
from jax import lax
from jax.experimental import pallas as pl
from jax.experimental.pallas import tpu as pltpu

B = 128
N = 8732
ROWS = 8


def _loss_body(ploc_ref, gloc_ref, pconf_ref, glabel_ref, dbox_ref, out_ref):
    step = pl.program_id(0)

    glabel = glabel_ref[...]
    mask = glabel > 0
    maskf = mask.astype(jnp.float32)
    num_pos = jnp.sum(maskf, axis=1, keepdims=True)

    loc_sum = jnp.zeros_like(num_pos)
    for c in range(4):
        g = gloc_ref[:, c, :]
        p = ploc_ref[:, c, :]
        d = dbox_ref[0, c, :][None, :]
        dwh = dbox_ref[0, c + 2 if c < 2 else c, :][None, :]
        if c < 2:
            off = (g - d) / dwh
        else:
            off = jnp.log(g / dwh)
        ad = jnp.abs(p - off)
        l1 = jnp.where(ad < 1.0, 0.5 * ad * ad, ad - 0.5)
        loc_sum = loc_sum + jnp.sum(maskf * l1, axis=1, keepdims=True)

    a = pconf_ref[:, 0, :]
    b = pconf_ref[:, 1, :]
    diff = a - b
    s = jnp.where(glabel == 0, -diff, diff)
    closs = jnp.maximum(s, 0.0) + jnp.log1p(jnp.exp(-jnp.abs(s)))

    pos_sum = jnp.sum(closs * maskf, axis=1, keepdims=True)
    con_neg = jnp.where(mask, 0.0, closs)
    neg_total = jnp.sum(con_neg, axis=1, keepdims=True)
    bits = lax.bitcast_convert_type(con_neg, jnp.int32)
    pcount = jnp.sum((con_neg > 0.0).astype(jnp.int32), axis=1, keepdims=True)

    npos_i = num_pos.astype(jnp.int32)
    K = jnp.minimum(3 * npos_i, N)

    def val_step(_, lohi):
        lo, hi = lohi
        mid = lo + (hi - lo) // 2
        cnt = jnp.sum((bits > mid).astype(jnp.int32), axis=1, keepdims=True)
        take = cnt < K
        return jnp.where(take, lo, mid + 1), jnp.where(take, mid, hi)

    lo0 = jnp.zeros_like(K)
    hi0 = jnp.full_like(K, (1 << 31) - 1)
    lo, hi = lax.fori_loop(0, 31, val_step, (lo0, hi0))
    v = lo
    thr = lax.bitcast_convert_type(v, jnp.float32)
    gt = bits > v
    cnt_gt = jnp.sum(gt.astype(jnp.int32), axis=1, keepdims=True)
    sum_gt = jnp.sum(jnp.where(gt, con_neg, 0.0), axis=1, keepdims=True)
    ties = jnp.maximum(K - cnt_gt, 0).astype(jnp.float32)
    sel_a = sum_gt + jnp.where(ties > 0, ties * thr, 0.0)

    iszero = con_neg == 0.0
    m_need = K - pcount
    idx = lax.broadcasted_iota(jnp.int32, iszero.shape, 1)

    def idx_step(_, lohi):
        lo, hi = lohi
        mid = lo + (hi - lo) // 2
        cz = jnp.sum((iszero & (idx < mid)).astype(jnp.int32), axis=1,
                     keepdims=True)
        take = cz >= m_need
        return jnp.where(take, lo, mid + 1), jnp.where(take, mid, hi)

    lo, hi = lax.fori_loop(0, 14, idx_step, (lo0, jnp.full_like(K, N)))
    pz = lo
    zero_part = jnp.sum(jnp.where(iszero & (idx < pz), closs, 0.0), axis=1,
                        keepdims=True)
    sel_b = neg_total + zero_part

    sel = jnp.where(K == 0, 0.0, jnp.where(K <= pcount, sel_a, sel_b))

    total = loc_sum + pos_sum + sel
    num_mask = (npos_i > 0).astype(jnp.float32)
    scaled = total * num_mask / jnp.maximum(num_pos, 1e-6)
    part = jnp.sum(scaled, axis=0, keepdims=True) * (1.0 / B)

    @pl.when(step == 0)
    def _():
        out_ref[...] = jnp.zeros((1, 1), jnp.float32)

    out_ref[...] += part


@jax.jit
def kernel(ploc, pconf, gloc, glabel, dboxes):
    gloc_t = jnp.swapaxes(gloc, 1, 2)
    glabel32 = glabel.astype(jnp.int32)
    dbox_t = jnp.swapaxes(dboxes, 0, 1)[None]

    grid = (B // ROWS,)
    out = pl.pallas_call(
        _loss_body,
        grid=grid,
        in_specs=[
            pl.BlockSpec((ROWS, 4, N), lambda i: (i, 0, 0)),
            pl.BlockSpec((ROWS, 4, N), lambda i: (i, 0, 0)),
            pl.BlockSpec((ROWS, 2, N), lambda i: (i, 0, 0)),
            pl.BlockSpec((ROWS, N), lambda i: (i, 0)),
            pl.BlockSpec((1, 4, N), lambda i: (0, 0, 0)),
        ],
        out_specs=pl.BlockSpec((1, 1), lambda i: (0, 0)),
        out_shape=jax.ShapeDtypeStruct((1, 1), jnp.float32),
        compiler_params=pltpu.CompilerParams(
            dimension_semantics=("arbitrary",)),
    )(ploc, gloc_t, pconf, glabel32, dbox_t)
    return out[0, 0]


# speedup vs baseline: 10.2070x; 1.6106x over previous
import functools

import jax
import jax.numpy as jnp
from jax import lax
from jax.experimental import pallas as pl
from jax.experimental.pallas import tpu as pltpu
from jax.experimental.pallas import tpu_sc as plsc

B = 128
N = 8732
NPAD = 8736
CHUNKS = NPAD // 16
NWORK = 32
RPW = B // NWORK
ROWS = 8


def _lane_iota():
    return lax.broadcasted_iota(jnp.int32, (16,), 0)


def _sc_body(pconf_hbm, glabel_hbm, out_hbm, a_v, b_v, gl_v, closs_v,
             conneg_v, out_v, sel_s, dma_sem):
    wid = lax.axis_index("s") * 2 + lax.axis_index("c")
    lanes = _lane_iota()
    conf_vec = jnp.zeros((16,), jnp.float32)

    for j in range(RPW):
        row = wid * RPW + j
        pltpu.async_copy(pconf_hbm.at[row, 0], a_v, dma_sem).wait()
        pltpu.async_copy(pconf_hbm.at[row, 1], b_v, dma_sem).wait()
        pltpu.async_copy(glabel_hbm.at[row], gl_v, dma_sem).wait()

        def ce_chunk(i, carry):
            tot, pos, npos, pcnt = carry
            sl = pl.ds(i * 16, 16)
            a = a_v[sl]
            b = b_v[sl]
            g = gl_v[sl]
            s = jnp.where(g == 0, b - a, a - b)
            u = jnp.exp(-jnp.abs(s))
            z = u / (2.0 + u)
            z2 = z * z
            lp = 2.0 * z * (1.0 + z2 * (1.0 / 3.0 + z2 * (
                1.0 / 5.0 + z2 * (1.0 / 7.0 + z2 * (1.0 / 9.0)))))
            closs = jnp.maximum(s, 0.0) + lp
            posm = g > 0
            cn = jnp.where(posm, 0.0, closs)
            closs_v[sl] = closs
            conneg_v[sl] = cn
            return (tot + closs,
                    pos + jnp.where(posm, closs, 0.0),
                    npos + posm.astype(jnp.int32),
                    pcnt + (cn > 0.0).astype(jnp.int32))

        zf = jnp.zeros((16,), jnp.float32)
        zi = jnp.zeros((16,), jnp.int32)
        tot, pos, npos, pcnt = lax.fori_loop(0, CHUNKS, ce_chunk,
                                             (zf, zf, zi, zi))
        tot_s = jnp.sum(tot)
        pos_s = jnp.sum(pos)
        npos_s = jnp.sum(npos)
        pcnt_s = jnp.sum(pcnt)
        K = jnp.minimum(3 * npos_s, N)

        @pl.when(K >= N)
        def _():
            sel_s[0] = tot_s

        @pl.when(jnp.logical_and(K < N, K <= pcnt_s))
        def _():
            def count_gt(t):
                def cbody(i, acc):
                    bits = plsc.bitcast(conneg_v[pl.ds(i * 16, 16)],
                                        jnp.int32)
                    return acc + (bits > t).astype(jnp.int32)
                return jnp.sum(lax.fori_loop(0, CHUNKS, cbody, zi))

            def vstep(_, lohi):
                lo, hi = lohi
                mid = lo + (hi - lo) // 2
                take = count_gt(mid) < K
                return (jnp.where(take, lo, mid + 1),
                        jnp.where(take, mid, hi))

            lo, hi = lax.fori_loop(0, 31, vstep,
                                   (jnp.int32(0), jnp.int32((1 << 31) - 1)))
            v = lo

            def gt_sums(i, carry):
                cnt, sm = carry
                cn = conneg_v[pl.ds(i * 16, 16)]
                gt = plsc.bitcast(cn, jnp.int32) > v
                return (cnt + gt.astype(jnp.int32),
                        sm + jnp.where(gt, cn, 0.0))

            cnt_gt, sum_gt = lax.fori_loop(0, CHUNKS, gt_sums, (zi, zf))
            cnt_gt_s = jnp.sum(cnt_gt)
            sum_gt_s = jnp.sum(sum_gt)
            thr_vec = plsc.bitcast(jnp.full((16,), v, jnp.int32),
                                   jnp.float32)
            thr_s = jnp.sum(jnp.where(lanes == 0, thr_vec, 0.0))
            ties = (K - cnt_gt_s).astype(jnp.float32)
            sel_s[0] = sum_gt_s + jnp.where(ties > 0, ties * thr_s, 0.0)

        @pl.when(jnp.logical_and(K < N, K > pcnt_s))
        def _():
            m = K - pcnt_s

            def count_zlt(p):
                def cbody(i, acc):
                    cn = conneg_v[pl.ds(i * 16, 16)]
                    idx = i * 16 + lanes
                    hit = jnp.logical_and(cn == 0.0, idx < p)
                    return acc + hit.astype(jnp.int32)
                return jnp.sum(lax.fori_loop(0, CHUNKS, cbody, zi))

            def istep(_, lohi):
                lo, hi = lohi
                mid = lo + (hi - lo) // 2
                take = count_zlt(mid) >= m
                return (jnp.where(take, lo, mid + 1),
                        jnp.where(take, mid, hi))

            lo, hi = lax.fori_loop(0, 14, istep,
                                   (jnp.int32(0), jnp.int32(N)))
            p = lo

            def zbody(i, acc):
                sl = pl.ds(i * 16, 16)
                cn = conneg_v[sl]
                idx = i * 16 + lanes
                hit = jnp.logical_and(cn == 0.0, idx < p)
                return acc + jnp.where(hit, closs_v[sl], 0.0)

            zero_part = jnp.sum(lax.fori_loop(0, CHUNKS, zbody, zf))
            sel_s[0] = (tot_s - pos_s) + zero_part

        sel = sel_s[0]
        conf_row = pos_s + sel
        conf_vec = jnp.where(lanes == j, conf_row, conf_vec)

    out_v[...] = conf_vec
    pltpu.sync_copy(out_v, out_hbm.at[wid])


def _sc_conf(pconf_pad, glabel_pad):
    mesh = plsc.VectorSubcoreMesh(core_axis_name="c", subcore_axis_name="s")
    kern = functools.partial(
        pl.kernel,
        out_type=jax.ShapeDtypeStruct((NWORK, 16), jnp.float32),
        mesh=mesh,
        scratch_types=[
            pltpu.VMEM((NPAD,), jnp.float32),
            pltpu.VMEM((NPAD,), jnp.float32),
            pltpu.VMEM((NPAD,), jnp.int32),
            pltpu.VMEM((NPAD,), jnp.float32),
            pltpu.VMEM((NPAD,), jnp.float32),
            pltpu.VMEM((16,), jnp.float32),
            pltpu.SMEM((1,), jnp.float32),
            pltpu.SemaphoreType.DMA,
        ],
        compiler_params=pltpu.CompilerParams(needs_layout_passes=False),
    )(_sc_body)
    return kern(pconf_pad, glabel_pad)


def _tc_body(ploc_ref, gloc_ref, glabel_ref, dbox_ref, conf_ref, out_ref):
    step = pl.program_id(0)

    glabel = glabel_ref[...]
    maskf = (glabel > 0).astype(jnp.float32)
    num_pos = jnp.sum(maskf, axis=1, keepdims=True)

    loc_sum = jnp.zeros_like(num_pos)
    for c in range(4):
        g = gloc_ref[:, c, :]
        p = ploc_ref[:, c, :]
        if c < 2:
            off = (g - dbox_ref[0, c, :][None, :]) / dbox_ref[0, c + 2, :][None, :]
        else:
            off = jnp.log(g / dbox_ref[0, c, :][None, :])
        ad = jnp.abs(p - off)
        l1 = jnp.where(ad < 1.0, 0.5 * ad * ad, ad - 0.5)
        loc_sum = loc_sum + jnp.sum(maskf * l1, axis=1, keepdims=True)

    conf = conf_ref[0]
    total = loc_sum + conf
    num_mask = (num_pos > 0).astype(jnp.float32)
    scaled = total * num_mask / jnp.maximum(num_pos, 1e-6)
    part = jnp.sum(scaled, axis=0, keepdims=True) * (1.0 / B)

    @pl.when(step == 0)
    def _():
        out_ref[...] = jnp.zeros((1, 1), jnp.float32)

    out_ref[...] += part


@jax.jit
def kernel(ploc, pconf, gloc, glabel, dboxes):
    glabel32 = glabel.astype(jnp.int32)

    padA = jnp.full((B, 1, NPAD - N), 40.0, jnp.float32)
    padB = jnp.full((B, 1, NPAD - N), -40.0, jnp.float32)
    pconf_pad = jnp.concatenate(
        [pconf, jnp.concatenate([padA, padB], axis=1)], axis=2)
    glabel_pad = jnp.concatenate(
        [glabel32, jnp.zeros((B, NPAD - N), jnp.int32)], axis=1)

    conf_sc = _sc_conf(pconf_pad, glabel_pad)
    conf3d = conf_sc[:, :RPW].reshape(B // ROWS, ROWS, 1)

    gloc_t = jnp.swapaxes(gloc, 1, 2)
    dbox_t = jnp.swapaxes(dboxes, 0, 1)[None]

    out = pl.pallas_call(
        _tc_body,
        grid=(B // ROWS,),
        in_specs=[
            pl.BlockSpec((ROWS, 4, N), lambda i: (i, 0, 0)),
            pl.BlockSpec((ROWS, 4, N), lambda i: (i, 0, 0)),
            pl.BlockSpec((ROWS, N), lambda i: (i, 0)),
            pl.BlockSpec((1, 4, N), lambda i: (0, 0, 0)),
            pl.BlockSpec((1, ROWS, 1), lambda i: (i, 0, 0)),
        ],
        out_specs=pl.BlockSpec((1, 1), lambda i: (0, 0)),
        out_shape=jax.ShapeDtypeStruct((1, 1), jnp.float32),
        compiler_params=pltpu.CompilerParams(
            dimension_semantics=("arbitrary",)),
    )(ploc, gloc_t, glabel32, dbox_t, conf3d)
    return out[0, 0]
